# Initial kernel scaffold; baseline (speedup 1.0000x reference)
import jax
import jax.numpy as jnp
from jax.experimental import pallas as pl


def kernel(x, edge_index, W1, b1, g1, be1, W2, b2, g2, be2, W3, b3):
    raise NotImplementedError("write your pallas kernel here")


# speedup vs baseline: 27.9374x; 27.9374x over previous
import functools

import jax
import jax.numpy as jnp
from jax import lax
from jax.experimental import pallas as pl
from jax.experimental.pallas import tpu as pltpu
from jax.experimental.pallas import tpu_sc as plsc

N = 10000
D = 128
E = 320000
NC = 2
NS = 16
NW = NC * NS
EPT = E // NW
CH = 128
NCHUNK = 80
PADE = NCHUNK * CH - EPT
N_PAD = 10240
RPT = N_PAD // NS
EPS = 1e-5


@functools.cache
def _get_sc_deg():
  mesh = plsc.VectorSubcoreMesh(core_axis_name="c", subcore_axis_name="s")

  @functools.partial(
      pl.kernel,
      out_type=jax.ShapeDtypeStruct((NC, N_PAD), jnp.float32),
      mesh=mesh,
      scratch_types=[
          pltpu.VMEM((NCHUNK, CH), jnp.int32),
          pltpu.VMEM((CH,), jnp.float32),
          pltpu.VMEM((RPT,), jnp.float32),
          pltpu.VMEM_SHARED((N_PAD,), jnp.float32),
      ],
  )
  def _sc_deg(dst_hbm, out_hbm, dst_v, ones_v, zbuf_v, acc_sh):
    c = lax.axis_index("c")
    s = lax.axis_index("s")
    wid = c * NS + s

    pltpu.sync_copy(dst_hbm.at[wid], dst_v)

    one16 = jnp.ones((16,), jnp.float32)
    z16 = jnp.zeros((16,), jnp.float32)
    for k in range(CH // 16):
      ones_v[pl.ds(k * 16, 16)] = one16

    def _zero(i, carry):
      zbuf_v[pl.ds(i * 16, 16)] = z16
      return carry

    lax.fori_loop(0, RPT // 16, _zero, 0)
    pltpu.sync_copy(zbuf_v, acc_sh.at[pl.ds(s * RPT, RPT)])
    plsc.subcore_barrier()

    def _chunk(j, carry):
      pltpu.sync_copy(ones_v, acc_sh.at[dst_v.at[j]], add=True)
      return carry

    lax.fori_loop(0, NCHUNK, _chunk, 0)
    plsc.subcore_barrier()

    pltpu.sync_copy(acc_sh.at[pl.ds(s * RPT, RPT)],
                    out_hbm.at[c, pl.ds(s * RPT, RPT)])

  return _sc_deg


@functools.cache
def _get_sc_agg():
  mesh = plsc.VectorSubcoreMesh(core_axis_name="c", subcore_axis_name="s")

  @functools.partial(
      pl.kernel,
      out_type=jax.ShapeDtypeStruct((NC, N_PAD, D), jnp.float32),
      mesh=mesh,
      scratch_types=[
          pltpu.VMEM((CH,), jnp.int32),
          pltpu.VMEM((CH,), jnp.int32),
          pltpu.VMEM((NCHUNK, CH), jnp.int32),
          pltpu.VMEM((CH, D), jnp.float32),
          pltpu.VMEM((CH, D), jnp.float32),
          pltpu.VMEM_SHARED((N_PAD, D), jnp.float32),
          pltpu.SemaphoreType.DMA,
          pltpu.SemaphoreType.DMA,
          pltpu.SemaphoreType.DMA,
          pltpu.SemaphoreType.DMA,
      ],
  )
  def _sc_agg(hs_hbm, src_hbm, dst_hbm, out_hbm,
              sidx_a, sidx_b, dst_v, rows_a, rows_b, acc_sh,
              sg_a, sg_b, si_a, si_b):
    c = lax.axis_index("c")
    s = lax.axis_index("s")
    wid = c * NS + s

    pltpu.sync_copy(dst_hbm.at[wid], dst_v)

    z16 = jnp.zeros((16,), jnp.float32)

    def _zero(i, carry):
      for k in range(D // 16):
        rows_a[i, pl.ds(k * 16, 16)] = z16
      return carry

    lax.fori_loop(0, CH, _zero, 0)
    for m in range(RPT // CH):
      pltpu.sync_copy(rows_a, acc_sh.at[pl.ds(s * RPT + m * CH, CH)])
    plsc.subcore_barrier()

    pltpu.sync_copy(src_hbm.at[wid, 0], sidx_a)
    pltpu.async_copy(hs_hbm.at[sidx_a], rows_a, sg_a)
    pltpu.async_copy(src_hbm.at[wid, 1], sidx_b, si_b)

    def _body(i, carry):
      ja = i * 2
      jb = ja + 1
      jn2 = jnp.minimum(ja + 2, NCHUNK - 1)
      jn3 = jnp.minimum(ja + 3, NCHUNK - 1)
      pltpu.make_async_copy(src_hbm.at[wid, jb], sidx_b, si_b).wait()
      pltpu.async_copy(hs_hbm.at[sidx_b], rows_b, sg_b)
      pltpu.make_async_copy(hs_hbm.at[sidx_a], rows_a, sg_a).wait()
      pltpu.async_copy(src_hbm.at[wid, jn2], sidx_a, si_a)
      pltpu.sync_copy(rows_a, acc_sh.at[dst_v.at[ja]], add=True)
      pltpu.make_async_copy(src_hbm.at[wid, jn2], sidx_a, si_a).wait()
      pltpu.async_copy(hs_hbm.at[sidx_a], rows_a, sg_a)
      pltpu.make_async_copy(hs_hbm.at[sidx_b], rows_b, sg_b).wait()
      pltpu.async_copy(src_hbm.at[wid, jn3], sidx_b, si_b)
      pltpu.sync_copy(rows_b, acc_sh.at[dst_v.at[jb]], add=True)
      return carry

    lax.fori_loop(0, NCHUNK // 2, _body, 0)
    pltpu.make_async_copy(hs_hbm.at[sidx_a], rows_a, sg_a).wait()
    pltpu.make_async_copy(src_hbm.at[wid, 0], sidx_b, si_b).wait()
    plsc.subcore_barrier()

    pltpu.sync_copy(acc_sh.at[pl.ds(s * RPT, RPT)],
                    out_hbm.at[c, pl.ds(s * RPT, RPT)])

  return _sc_agg


def _tc_dinv(partials):
  def body(p_ref, o_ref):
    p = p_ref[...]
    deg = p[0:1, :] + p[1:2, :] + 1.0
    col = lax.broadcasted_iota(jnp.int32, (1, N_PAD), 1)
    o_ref[...] = jnp.where(col < N, lax.rsqrt(deg), 0.0)

  return pl.pallas_call(
      body, out_shape=jax.ShapeDtypeStruct((1, N_PAD), jnp.float32)
  )(partials)


def _tc_pre(x_pad, W, dcol):
  def body(x_ref, w_ref, d_ref, o_ref):
    h = jnp.dot(x_ref[...], w_ref[...], preferred_element_type=jnp.float32)
    o_ref[...] = h * d_ref[...]

  return pl.pallas_call(
      body, out_shape=jax.ShapeDtypeStruct((N_PAD, D), jnp.float32)
  )(x_pad, W, dcol)


def _tc_mid(p0, p1, hs, dcol, b, g, be, Wn):
  def body(p0r, p1r, hsr, dr, br, gr, ber, wr, o_ref):
    t = (p0r[...] + p1r[...] + hsr[...]) * dr[...] + br[...]
    row = lax.broadcasted_iota(jnp.int32, (N_PAD, 1), 0)
    m = jnp.where(row < N, 1.0, 0.0)
    mean = jnp.sum(t * m, axis=0, keepdims=True) * (1.0 / N)
    ctr = (t - mean) * m
    var = jnp.sum(ctr * ctr, axis=0, keepdims=True) * (1.0 / N)
    y = jnp.maximum(gr[...] * ctr * lax.rsqrt(var + EPS) + ber[...], 0.0)
    h = jnp.dot(y, wr[...], preferred_element_type=jnp.float32)
    o_ref[...] = h * dr[...]

  return pl.pallas_call(
      body, out_shape=jax.ShapeDtypeStruct((N_PAD, D), jnp.float32)
  )(p0, p1, hs, dcol, b, g, be, Wn)


def _tc_post(p0, p1, hs, dcol, b):
  def body(p0r, p1r, hsr, dr, br, o_ref):
    o_ref[...] = (p0r[...] + p1r[...] + hsr[...]) * dr[...] + br[...]

  return pl.pallas_call(
      body, out_shape=jax.ShapeDtypeStruct((N_PAD, D), jnp.float32)
  )(p0, p1, hs, dcol, b)


def kernel(x, edge_index, W1, b1, g1, be1, W2, b2, g2, be2, W3, b3):
  ei = edge_index.astype(jnp.int32)
  src = ei[0].reshape(NW, EPT)
  dst = ei[1].reshape(NW, EPT)

  i = jnp.arange(PADE, dtype=jnp.int32)[None, :]
  w = jnp.arange(NW, dtype=jnp.int32)[:, None]
  src_pad = (i * NW + w) % N
  dst_pad = N + (i * 7 + w) % PADE
  src3 = jnp.concatenate([src, src_pad], axis=1).reshape(NW, NCHUNK, CH)
  dst3 = jnp.concatenate([dst, dst_pad], axis=1).reshape(NW, NCHUNK, CH)

  x_pad = jnp.concatenate(
      [x, jnp.zeros((N_PAD - N, D), dtype=x.dtype)], axis=0)

  sc_deg = _get_sc_deg()
  sc_agg = _get_sc_agg()

  degp = sc_deg(dst3)
  dinv = _tc_dinv(degp)
  dcol = dinv.reshape(N_PAD, 1)

  b1r, g1r, be1r = b1.reshape(1, D), g1.reshape(1, D), be1.reshape(1, D)
  b2r, g2r, be2r = b2.reshape(1, D), g2.reshape(1, D), be2.reshape(1, D)
  b3r = b3.reshape(1, D)

  hs1 = _tc_pre(x_pad, W1, dcol)
  p = sc_agg(hs1, src3, dst3)
  hs2 = _tc_mid(p[0], p[1], hs1, dcol, b1r, g1r, be1r, W2)
  p = sc_agg(hs2, src3, dst3)
  hs3 = _tc_mid(p[0], p[1], hs2, dcol, b2r, g2r, be2r, W3)
  p = sc_agg(hs3, src3, dst3)
  out = _tc_post(p[0], p[1], hs3, dcol, b3r)
  return out[:N]
